# Initial kernel scaffold; baseline (speedup 1.0000x reference)
import jax
import jax.numpy as jnp
from jax.experimental import pallas as pl


def kernel(x, codes, codebooks, scales, bias):
    raise NotImplementedError("write your pallas kernel here")


# speedup vs baseline: 25.4115x; 25.4115x over previous
import dataclasses
import functools

import jax
import jax.numpy as jnp
from jax import lax
from jax.experimental import pallas as pl
from jax.experimental.pallas import tpu as pltpu
from jax.experimental.pallas import tpu_sc as plsc

NUM_CORES = 2
NUM_SUBCORES = 16
NW = NUM_CORES * NUM_SUBCORES
L = 16

IN_F = 2048
OUT_F = 2048
IN_GROUP = 8
GROUPS = IN_F // IN_GROUP
ROWS_PER_W = OUT_F // NW
CHUNK_R = 16
N_CHUNKS = ROWS_PER_W // CHUNK_R


def _dequant_body(idx0_hbm, idx1_hbm, table_hbm, scales_hbm, w_hbm,
                  idx0_v, idx1_v, table_v, scales_v, wv):
    cid = lax.axis_index("c")
    sid = lax.axis_index("s")
    wid = sid * NUM_CORES + cid
    row0 = wid * ROWS_PER_W

    pltpu.sync_copy(table_hbm, table_v)
    pltpu.sync_copy(scales_hbm.at[pl.ds(row0 * 1, ROWS_PER_W)], scales_v)

    iota = lax.iota(jnp.int32, L)
    rep = iota >> 3
    cols = iota & 7

    @pl.loop(0, N_CHUNKS)
    def _chunk(c):
        base_row = row0 + c * CHUNK_R
        pltpu.sync_copy(
            idx0_hbm.at[pl.ds(base_row * GROUPS, CHUNK_R * GROUPS)], idx0_v)
        pltpu.sync_copy(
            idx1_hbm.at[pl.ds(base_row * GROUPS, CHUNK_R * GROUPS)], idx1_v)

        @pl.loop(0, CHUNK_R)
        def _row(r):
            lrow = c * CHUNK_R + r
            scale_vec = plsc.load_gather(
                scales_v, [jnp.broadcast_to(lrow, (L,))])

            @pl.loop(0, GROUPS // 2)
            def _pair(k):
                pat = rep + (r * GROUPS + 2 * k)
                r0 = plsc.load_gather(idx0_v, [pat])
                r1 = plsc.load_gather(idx1_v, [pat])
                a = plsc.load_gather(table_v, [r0, cols])
                b = plsc.load_gather(table_v, [r1, cols])
                w = (a + b) * scale_vec
                wv[pl.ds(r * IN_F + k * L, L)] = w

        pltpu.sync_copy(wv, w_hbm.at[pl.ds(base_row * IN_F, CHUNK_R * IN_F)])


def _dequant_sc(idx0, idx1, table, scales_flat):
    mesh = plsc.VectorSubcoreMesh(core_axis_name="c", subcore_axis_name="s")
    cp = pltpu.CompilerParams()
    if "needs_layout_passes" in pltpu.CompilerParams.__dataclass_fields__:
        cp = dataclasses.replace(cp, needs_layout_passes=False)
    f = pl.kernel(
        _dequant_body,
        out_type=jax.ShapeDtypeStruct((OUT_F * IN_F,), jnp.float32),
        mesh=mesh,
        scratch_types=[
            pltpu.VMEM((CHUNK_R * GROUPS,), jnp.int32),
            pltpu.VMEM((CHUNK_R * GROUPS,), jnp.int32),
            pltpu.VMEM((2 * 256, IN_GROUP), jnp.float32),
            pltpu.VMEM((ROWS_PER_W,), jnp.float32),
            pltpu.VMEM((CHUNK_R * IN_F,), jnp.float32),
        ],
        compiler_params=cp,
    )
    return f(idx0, idx1, table, scales_flat)


BN = 512


def _mm_body(x_ref, w_ref, b_ref, o_ref):
    o_ref[...] = lax.dot_general(
        x_ref[...], w_ref[...], (((1,), (1,)), ((), ())),
        preferred_element_type=jnp.float32) + b_ref[...]


def _matmul_tc(x2, w, bias2):
    return pl.pallas_call(
        _mm_body,
        grid=(OUT_F // BN,),
        in_specs=[
            pl.BlockSpec((x2.shape[0], IN_F), lambda j: (0, 0)),
            pl.BlockSpec((BN, IN_F), lambda j: (j, 0)),
            pl.BlockSpec((1, BN), lambda j: (0, j)),
        ],
        out_specs=pl.BlockSpec((x2.shape[0], BN), lambda j: (0, j)),
        out_shape=jax.ShapeDtypeStruct((x2.shape[0], OUT_F), jnp.float32),
    )(x2, w, bias2)


def kernel(x, codes, codebooks, scales, bias):
    idx0 = codes[:, :, 0].reshape(-1)
    idx1 = codes[:, :, 1].reshape(-1) + jnp.int32(256)
    table = codebooks.reshape(2 * 256, IN_GROUP)
    scales_flat = scales.reshape(OUT_F)
    wflat = _dequant_sc(idx0, idx1, table, scales_flat)
    w = wflat.reshape(OUT_F, IN_F)
    seq = x.shape[0] * x.shape[1]
    out = _matmul_tc(x.reshape(seq, IN_F), w, bias.reshape(1, OUT_F))
    return out.reshape(x.shape[0], x.shape[1], OUT_F)
